# Initial kernel scaffold; baseline (speedup 1.0000x reference)
import jax
import jax.numpy as jnp
from jax.experimental import pallas as pl


def kernel(x, M, T, b, raw_slopes):
    raise NotImplementedError("write your pallas kernel here")


# speedup vs baseline: 34.8126x; 34.8126x over previous
import functools

import jax
import jax.numpy as jnp
from jax import lax
from jax.experimental import pallas as pl
from jax.experimental.pallas import tpu as pltpu
from jax.experimental.pallas import tpu_sc as plsc

KSEG = 16
NC, NS = 2, 16
NW = NC * NS
TOTAL = 32 * 512 * 512 * 3
PER_W = TOTAL // NW
CHUNK = 24576
NCHUNK = PER_W // CHUNK
BODIES = CHUNK // 48


def _isp_body(x_hbm, params_hbm, slopes_hbm, out_hbm,
              inbuf, outbuf, params_v, slopes_v,
              a0, a1, a2, b0, b1, b2):
    f32 = jnp.float32
    wid = lax.axis_index("s") * NC + lax.axis_index("c")
    wbase = wid * PER_W

    pltpu.sync_copy(params_hbm, params_v)
    pltpu.sync_copy(slopes_hbm, slopes_v)

    iota_i = lax.iota(jnp.int32, KSEG)
    knots = iota_i.astype(f32) * (1.0 / KSEG)

    for c, (at, bt) in enumerate(((a0, b0), (a1, b1), (a2, b2))):
        r = slopes_v[pl.ds(c * KSEG, KSEG)]
        e = jnp.exp(r - jnp.max(r))
        s_vec = jnp.broadcast_to(jnp.sum(e), (KSEG,))
        slope = e * (KSEG / s_vec)
        h = slope * (1.0 / KSEG)
        cum_ex = plsc.cumsum(h) - h
        at[...] = slope
        bt[...] = cum_ex - slope * knots

    pv = params_v[...]
    coef = []
    for i in range(3):
        t_i = pv[9 + i]
        coef.append(tuple(t_i * pv[3 * i + j] for j in range(3)))
    bias = tuple(pv[12 + i] for i in range(3))

    iota3 = iota_i * 3

    def chunk_step(k, _):
        off = wbase + k * CHUNK
        pltpu.sync_copy(x_hbm.at[pl.ds(off, CHUNK)], inbuf)

        def body(j, _):
            base = j * 48
            i0 = iota3 + base
            i1 = i0 + 1
            i2 = i0 + 2
            r = plsc.load_gather(inbuf, [i0])
            g = plsc.load_gather(inbuf, [i1])
            bl = plsc.load_gather(inbuf, [i2])
            for c, (at, bt, ic) in enumerate(((a0, b0, i0),
                                             (a1, b1, i1),
                                             (a2, b2, i2))):
                y = coef[c][0] * r + coef[c][1] * g + coef[c][2] * bl + bias[c]
                y = jnp.clip(y, 0.0, 1.0)
                idx = jnp.minimum((y * KSEG).astype(jnp.int32), KSEG - 1)
                av = plsc.load_gather(at, [idx])
                bv = plsc.load_gather(bt, [idx])
                f = jnp.clip(av * y + bv, 0.0, 1.0)
                plsc.store_scatter(outbuf, [ic], f)
            return 0

        lax.fori_loop(0, BODIES, body, 0)
        pltpu.sync_copy(outbuf, out_hbm.at[pl.ds(off, CHUNK)])
        return 0

    lax.fori_loop(0, NCHUNK, chunk_step, 0)


_mesh = plsc.VectorSubcoreMesh(core_axis_name="c", subcore_axis_name="s",
                               num_cores=NC, num_subcores=NS)

_isp = functools.partial(
    pl.kernel,
    out_type=jax.ShapeDtypeStruct((TOTAL,), jnp.float32),
    mesh=_mesh,
    compiler_params=pltpu.CompilerParams(needs_layout_passes=False),
    scratch_types=[
        pltpu.VMEM((CHUNK,), jnp.float32),
        pltpu.VMEM((CHUNK,), jnp.float32),
        pltpu.VMEM((16,), jnp.float32),
        pltpu.VMEM((48,), jnp.float32),
        pltpu.VMEM((KSEG,), jnp.float32),
        pltpu.VMEM((KSEG,), jnp.float32),
        pltpu.VMEM((KSEG,), jnp.float32),
        pltpu.VMEM((KSEG,), jnp.float32),
        pltpu.VMEM((KSEG,), jnp.float32),
        pltpu.VMEM((KSEG,), jnp.float32),
    ],
)(_isp_body)


@jax.jit
def kernel(x, M, T, b, raw_slopes):
    xf = x.reshape(-1)
    params = jnp.concatenate(
        [M.reshape(-1), T, b, jnp.zeros((1,), jnp.float32)])
    slopes_t = raw_slopes.T.reshape(-1)
    out = _isp(xf, params, slopes_t)
    return out.reshape(x.shape)


# speedup vs baseline: 35.7610x; 1.0272x over previous
import functools

import jax
import jax.numpy as jnp
from jax import lax
from jax.experimental import pallas as pl
from jax.experimental.pallas import tpu as pltpu
from jax.experimental.pallas import tpu_sc as plsc

KSEG = 16
NC, NS = 2, 16
NW = NC * NS
TOTAL = 32 * 512 * 512 * 3
PER_W = TOTAL // NW
CHUNK = 24576
NCHUNK = PER_W // CHUNK
BODIES = CHUNK // 48


def _isp_body(x_hbm, params_hbm, slopes_hbm, out_hbm,
              inbuf, outbuf, params_v, slopes_v,
              a0, a1, a2, b0, b1, b2):
    f32 = jnp.float32
    wid = lax.axis_index("s") * NC + lax.axis_index("c")
    wbase = wid * PER_W

    pltpu.sync_copy(params_hbm, params_v)
    pltpu.sync_copy(slopes_hbm, slopes_v)

    iota_i = lax.iota(jnp.int32, KSEG)
    knots = iota_i.astype(f32) * (1.0 / KSEG)

    for c, (at, bt) in enumerate(((a0, b0), (a1, b1), (a2, b2))):
        r = slopes_v[pl.ds(c * KSEG, KSEG)]
        e = jnp.exp(r - jnp.max(r))
        s_vec = jnp.broadcast_to(jnp.sum(e), (KSEG,))
        slope = e * (KSEG / s_vec)
        h = slope * (1.0 / KSEG)
        cum_ex = plsc.cumsum(h) - h
        at[...] = slope
        bt[...] = cum_ex - slope * knots

    pv = params_v[...]
    coef = []
    for i in range(3):
        t_i = pv[9 + i]
        coef.append(tuple(t_i * pv[3 * i + j] for j in range(3)))
    bias = tuple(pv[12 + i] for i in range(3))

    iota3 = iota_i * 3

    def chunk_step(k, _):
        off = wbase + k * CHUNK
        pltpu.sync_copy(x_hbm.at[pl.ds(off, CHUNK)], inbuf)

        @plsc.parallel_loop(0, CHUNK, step=48, unroll=4)
        def body(base):
            i0 = iota3 + base
            i1 = i0 + 1
            i2 = i0 + 2
            r = plsc.load_gather(inbuf, [i0])
            g = plsc.load_gather(inbuf, [i1])
            bl = plsc.load_gather(inbuf, [i2])
            for c, (at, bt, ic) in enumerate(((a0, b0, i0),
                                             (a1, b1, i1),
                                             (a2, b2, i2))):
                y = coef[c][0] * r + coef[c][1] * g + coef[c][2] * bl + bias[c]
                y = jnp.clip(y, 0.0, 1.0)
                idx = jnp.minimum((y * KSEG).astype(jnp.int32), KSEG - 1)
                av = plsc.load_gather(at, [idx])
                bv = plsc.load_gather(bt, [idx])
                f = jnp.clip(av * y + bv, 0.0, 1.0)
                plsc.store_scatter(outbuf, [ic], f)

        pltpu.sync_copy(outbuf, out_hbm.at[pl.ds(off, CHUNK)])
        return 0

    lax.fori_loop(0, NCHUNK, chunk_step, 0)


_mesh = plsc.VectorSubcoreMesh(core_axis_name="c", subcore_axis_name="s",
                               num_cores=NC, num_subcores=NS)

_isp = functools.partial(
    pl.kernel,
    out_type=jax.ShapeDtypeStruct((TOTAL,), jnp.float32),
    mesh=_mesh,
    compiler_params=pltpu.CompilerParams(needs_layout_passes=False),
    scratch_types=[
        pltpu.VMEM((CHUNK,), jnp.float32),
        pltpu.VMEM((CHUNK,), jnp.float32),
        pltpu.VMEM((16,), jnp.float32),
        pltpu.VMEM((48,), jnp.float32),
        pltpu.VMEM((KSEG,), jnp.float32),
        pltpu.VMEM((KSEG,), jnp.float32),
        pltpu.VMEM((KSEG,), jnp.float32),
        pltpu.VMEM((KSEG,), jnp.float32),
        pltpu.VMEM((KSEG,), jnp.float32),
        pltpu.VMEM((KSEG,), jnp.float32),
    ],
)(_isp_body)


@jax.jit
def kernel(x, M, T, b, raw_slopes):
    xf = x.reshape(-1)
    params = jnp.concatenate(
        [M.reshape(-1), T, b, jnp.zeros((1,), jnp.float32)])
    slopes_t = raw_slopes.T.reshape(-1)
    out = _isp(xf, params, slopes_t)
    return out.reshape(x.shape)
